# baseline (speedup 1.0000x reference)
import functools

import jax
import jax.numpy as jnp
from jax import lax
from jax.experimental import pallas as pl
from jax.experimental.pallas import tpu as pltpu
from jax.experimental.pallas import tpu_sc as plsc

V = 1_000_000
E = 64
B = 16384
L = 200
PAD = 16

NC, NS = 2, 16
NW = NC * NS
ROWS_W = B // NW
EX_BLK = 4
GW = 100
NG = EX_BLK * L // GW
SB_EX = 16
SB_BLKS = SB_EX // EX_BLK
NSB = ROWS_W // SB_EX
SB_ROWS = SB_EX * L // GW

TBLK = 512
PGRID = (V + TBLK - 1) // TBLK

_mesh = plsc.VectorSubcoreMesh(core_axis_name="c", subcore_axis_name="s")


def _proj_body(t_ref, w1_ref, w2_ref, p_ref):
    w12 = jnp.dot(w1_ref[...], w2_ref[...],
                  preferred_element_type=jnp.float32) * (1.0 / L)
    p_ref[...] = lax.dot_general(t_ref[...], w12, (((0,), (0,)), ((), ())),
                                 preferred_element_type=jnp.float32)


def _project(tT, w1p, w2p):
    return pl.pallas_call(
        _proj_body,
        grid=(PGRID,),
        in_specs=[
            pl.BlockSpec((E, TBLK), lambda i: (0, i)),
            pl.BlockSpec((E, PAD), lambda i: (0, 0)),
            pl.BlockSpec((PAD, PAD), lambda i: (0, 0)),
        ],
        out_specs=pl.BlockSpec((TBLK, PAD), lambda i: (i, 0)),
        out_shape=jax.ShapeDtypeStruct((V, PAD), jnp.float32),
    )(tT, w1p, w2p)


@functools.partial(
    pl.kernel,
    out_type=jax.ShapeDtypeStruct((B, PAD), jnp.float32),
    mesh=_mesh,
    scratch_types=[
        pltpu.VMEM((2, SB_ROWS, GW), jnp.int32),
        pltpu.VMEM((2, EX_BLK * L, PAD), jnp.float32),
        pltpu.VMEM((2, EX_BLK, PAD), jnp.float32),
        pltpu.VMEM((PAD,), jnp.float32),
        pltpu.SemaphoreType.DMA,
        pltpu.SemaphoreType.DMA,
        pltpu.SemaphoreType.DMA,
        pltpu.SemaphoreType.DMA,
        pltpu.SemaphoreType.DMA,
        pltpu.SemaphoreType.DMA,
    ],
    compiler_params=pltpu.CompilerParams(use_tc_tiling_on_sc=False),
)
def _pool(x_hbm, p_hbm, b_hbm, z_hbm, idx_v, rows_v, zstage, bias_v,
          isem0, isem1, gsem0, gsem1, zsem0, zsem1):
    wid = lax.axis_index("c") * NS + lax.axis_index("s")
    isems = (isem0, isem1)
    gsems = (gsem0, gsem1)
    zsems = (zsem0, zsem1)

    def fire(ib, q, p):
        for j in range(NG):
            pltpu.async_copy(p_hbm.at[idx_v.at[ib, q * NG + j]],
                             rows_v.at[p, pl.ds(j * GW, GW)], gsems[p])

    def drain(ib, q, p):
        for j in range(NG):
            pltpu.make_async_copy(p_hbm.at[idx_v.at[ib, q * NG + j]],
                                  rows_v.at[p, pl.ds(j * GW, GW)],
                                  gsems[p]).wait()

    pltpu.sync_copy(b_hbm, bias_v)
    pltpu.sync_copy(x_hbm.at[wid * NSB], idx_v.at[0])
    fire(0, 0, 0)

    def outer(hh, carry):
        for ib in (0, 1):
            sb = hh * 2 + ib
            nib = 1 - ib

            @pl.when(sb + 1 < NSB)
            def _():
                pltpu.async_copy(x_hbm.at[wid * NSB + sb + 1],
                                 idx_v.at[nib], isems[nib])

            for q in range(SB_BLKS):
                p = q % 2
                np_ = 1 - p
                if q + 1 < SB_BLKS:
                    fire(ib, q + 1, np_)
                else:
                    @pl.when(sb + 1 < NSB)
                    def _():
                        pltpu.make_async_copy(
                            x_hbm.at[wid * NSB + sb + 1], idx_v.at[nib],
                            isems[nib]).wait()
                        fire(nib, 0, np_)
                drain(ib, q, p)

                zero = jnp.zeros((PAD,), jnp.float32)
                bias = bias_v[...]

                def example(r, c):
                    base = r * L

                    def acc_body(i, accs):
                        a0, a1, a2, a3 = accs
                        row = base + i * 4
                        a0 = a0 + rows_v[p, row, pl.ds(0, PAD)]
                        a1 = a1 + rows_v[p, row + 1, pl.ds(0, PAD)]
                        a2 = a2 + rows_v[p, row + 2, pl.ds(0, PAD)]
                        a3 = a3 + rows_v[p, row + 3, pl.ds(0, PAD)]
                        return (a0, a1, a2, a3)

                    a0, a1, a2, a3 = lax.fori_loop(
                        0, L // 4, acc_body, (zero, zero, zero, zero))
                    zstage[p, r, pl.ds(0, PAD)] = ((a0 + a1) + (a2 + a3)
                                                   ) + bias
                    return c

                blk_id = sb * SB_BLKS + q
                row0 = wid * ROWS_W + sb * SB_EX + q * EX_BLK

                @pl.when(blk_id >= 2)
                def _():
                    pltpu.make_async_copy(
                        zstage.at[p],
                        z_hbm.at[pl.ds(row0 - 2 * EX_BLK, EX_BLK)],
                        zsems[p]).wait()

                lax.fori_loop(0, EX_BLK, example, 0)
                pltpu.async_copy(zstage.at[p],
                                 z_hbm.at[pl.ds(row0, EX_BLK)], zsems[p])
        return carry

    lax.fori_loop(0, NSB // 2, outer, 0)
    last = wid * ROWS_W + ROWS_W - EX_BLK
    pltpu.make_async_copy(zstage.at[1],
                          z_hbm.at[pl.ds(last, EX_BLK)], zsems[1]).wait()
    pltpu.make_async_copy(zstage.at[0],
                          z_hbm.at[pl.ds(last - EX_BLK, EX_BLK)],
                          zsems[0]).wait()


def kernel(x, table, W1, b1, W2, b2):
    w1p = jnp.pad(W1, ((0, 0), (0, PAD - W1.shape[1])))
    w2p = jnp.pad(W2, ((0, PAD - W2.shape[0]), (0, PAD - W2.shape[1])))
    bias16 = jnp.pad(jnp.dot(b1, W2) + b2, (0, PAD - W2.shape[1]))
    proj = _project(table.T, w1p, w2p)
    x3 = x.astype(jnp.int32).reshape(B // SB_EX, SB_ROWS, GW)
    z = _pool(x3, proj, bias16)
    return z[:, : W2.shape[1]]


# speedup vs baseline: 2.2483x; 2.2483x over previous
import functools

import jax
import jax.numpy as jnp
from jax import lax
from jax.experimental import pallas as pl
from jax.experimental.pallas import tpu as pltpu
from jax.experimental.pallas import tpu_sc as plsc

V = 1_000_000
E = 64
B = 16384
L = 200
PAD = 16

NC, NS = 2, 16
NW = NC * NS
ROWS_W = B // NW
EX_BLK = 4
GW = 100
NG = EX_BLK * L // GW
SB_EX = 16
SB_BLKS = SB_EX // EX_BLK
NSB = ROWS_W // SB_EX
SB_ROWS = SB_EX * L // GW

TBLK = 8192
PGRID = (V + TBLK - 1) // TBLK

_mesh = plsc.VectorSubcoreMesh(core_axis_name="c", subcore_axis_name="s")


def _proj_body(t_ref, w1_ref, w2_ref, p_ref):
    w12 = jnp.dot(w1_ref[...], w2_ref[...],
                  preferred_element_type=jnp.float32) * (1.0 / L)
    p_ref[...] = lax.dot_general(t_ref[...], w12, (((0,), (0,)), ((), ())),
                                 preferred_element_type=jnp.float32)


def _project(tT, w1p, w2p):
    return pl.pallas_call(
        _proj_body,
        grid=(PGRID,),
        in_specs=[
            pl.BlockSpec((E, TBLK), lambda i: (0, i)),
            pl.BlockSpec((E, PAD), lambda i: (0, 0)),
            pl.BlockSpec((PAD, PAD), lambda i: (0, 0)),
        ],
        out_specs=pl.BlockSpec((TBLK, PAD), lambda i: (i, 0)),
        out_shape=jax.ShapeDtypeStruct((V, PAD), jnp.float32),
    )(tT, w1p, w2p)


@functools.partial(
    pl.kernel,
    out_type=jax.ShapeDtypeStruct((B, PAD), jnp.float32),
    mesh=_mesh,
    scratch_types=[
        pltpu.VMEM((2, SB_ROWS, GW), jnp.int32),
        pltpu.VMEM((2, EX_BLK * L, PAD), jnp.float32),
        pltpu.VMEM((2, EX_BLK, PAD), jnp.float32),
        pltpu.VMEM((PAD,), jnp.float32),
        pltpu.SemaphoreType.DMA,
        pltpu.SemaphoreType.DMA,
        pltpu.SemaphoreType.DMA,
        pltpu.SemaphoreType.DMA,
        pltpu.SemaphoreType.DMA,
        pltpu.SemaphoreType.DMA,
    ],
    compiler_params=pltpu.CompilerParams(use_tc_tiling_on_sc=False),
)
def _pool(x_hbm, p_hbm, b_hbm, z_hbm, idx_v, rows_v, zstage, bias_v,
          isem0, isem1, gsem0, gsem1, zsem0, zsem1):
    wid = lax.axis_index("c") * NS + lax.axis_index("s")
    isems = (isem0, isem1)
    gsems = (gsem0, gsem1)
    zsems = (zsem0, zsem1)

    def fire(ib, q, p):
        for j in range(NG):
            pltpu.async_copy(p_hbm.at[idx_v.at[ib, q * NG + j]],
                             rows_v.at[p, pl.ds(j * GW, GW)], gsems[p])

    def drain(ib, q, p):
        for j in range(NG):
            pltpu.make_async_copy(p_hbm.at[idx_v.at[ib, q * NG + j]],
                                  rows_v.at[p, pl.ds(j * GW, GW)],
                                  gsems[p]).wait()

    pltpu.sync_copy(b_hbm, bias_v)
    pltpu.sync_copy(x_hbm.at[wid * NSB], idx_v.at[0])
    fire(0, 0, 0)

    def outer(hh, carry):
        for ib in (0, 1):
            sb = hh * 2 + ib
            nib = 1 - ib

            @pl.when(sb + 1 < NSB)
            def _():
                pltpu.async_copy(x_hbm.at[wid * NSB + sb + 1],
                                 idx_v.at[nib], isems[nib])

            for q in range(SB_BLKS):
                p = q % 2
                np_ = 1 - p
                if q + 1 < SB_BLKS:
                    fire(ib, q + 1, np_)
                else:
                    @pl.when(sb + 1 < NSB)
                    def _():
                        pltpu.make_async_copy(
                            x_hbm.at[wid * NSB + sb + 1], idx_v.at[nib],
                            isems[nib]).wait()
                        fire(nib, 0, np_)
                drain(ib, q, p)

                zero = jnp.zeros((PAD,), jnp.float32)
                bias = bias_v[...]

                def example(r, c):
                    base = r * L

                    def acc_body(i, accs):
                        a0, a1, a2, a3 = accs
                        row = base + i * 4
                        a0 = a0 + rows_v[p, row, pl.ds(0, PAD)]
                        a1 = a1 + rows_v[p, row + 1, pl.ds(0, PAD)]
                        a2 = a2 + rows_v[p, row + 2, pl.ds(0, PAD)]
                        a3 = a3 + rows_v[p, row + 3, pl.ds(0, PAD)]
                        return (a0, a1, a2, a3)

                    a0, a1, a2, a3 = lax.fori_loop(
                        0, L // 4, acc_body, (zero, zero, zero, zero))
                    zstage[p, r, pl.ds(0, PAD)] = ((a0 + a1) + (a2 + a3)
                                                   ) + bias
                    return c

                blk_id = sb * SB_BLKS + q
                row0 = wid * ROWS_W + sb * SB_EX + q * EX_BLK

                @pl.when(blk_id >= 2)
                def _():
                    pltpu.make_async_copy(
                        zstage.at[p],
                        z_hbm.at[pl.ds(row0 - 2 * EX_BLK, EX_BLK)],
                        zsems[p]).wait()

                lax.fori_loop(0, EX_BLK, example, 0)
                pltpu.async_copy(zstage.at[p],
                                 z_hbm.at[pl.ds(row0, EX_BLK)], zsems[p])
        return carry

    lax.fori_loop(0, NSB // 2, outer, 0)
    last = wid * ROWS_W + ROWS_W - EX_BLK
    pltpu.make_async_copy(zstage.at[1],
                          z_hbm.at[pl.ds(last, EX_BLK)], zsems[1]).wait()
    pltpu.make_async_copy(zstage.at[0],
                          z_hbm.at[pl.ds(last - EX_BLK, EX_BLK)],
                          zsems[0]).wait()


def kernel(x, table, W1, b1, W2, b2):
    w1p = jnp.pad(W1, ((0, 0), (0, PAD - W1.shape[1])))
    w2p = jnp.pad(W2, ((0, PAD - W2.shape[0]), (0, PAD - W2.shape[1])))
    bias16 = jnp.pad(jnp.dot(b1, W2) + b2, (0, PAD - W2.shape[1]))
    proj = _project(table.T, w1p, w2p)
    x3 = x.astype(jnp.int32).reshape(B // SB_EX, SB_ROWS, GW)
    z = _pool(x3, proj, bias16)
    return z[:, : W2.shape[1]]


# speedup vs baseline: 2.3420x; 1.0417x over previous
import functools

import jax
import jax.numpy as jnp
from jax import lax
from jax.experimental import pallas as pl
from jax.experimental.pallas import tpu as pltpu
from jax.experimental.pallas import tpu_sc as plsc

V = 1_000_000
E = 64
B = 16384
L = 200
PAD = 16

NC, NS = 2, 16
NW = NC * NS
ROWS_W = B // NW
EX_BLK = 4
GW = 100
NG = EX_BLK * L // GW
SB_EX = 16
SB_BLKS = SB_EX // EX_BLK
NSB = ROWS_W // SB_EX
SB_ROWS = SB_EX * L // GW

TBLK = 32768
PGRID = (V + TBLK - 1) // TBLK

_mesh = plsc.VectorSubcoreMesh(core_axis_name="c", subcore_axis_name="s")


def _proj_body(t_ref, w1_ref, w2_ref, p_ref):
    w12 = jnp.dot(w1_ref[...], w2_ref[...],
                  preferred_element_type=jnp.float32) * (1.0 / L)
    p_ref[...] = lax.dot_general(t_ref[...], w12, (((0,), (0,)), ((), ())),
                                 preferred_element_type=jnp.float32)


def _project(tT, w1p, w2p):
    return pl.pallas_call(
        _proj_body,
        grid=(PGRID,),
        in_specs=[
            pl.BlockSpec((E, TBLK), lambda i: (0, i)),
            pl.BlockSpec((E, PAD), lambda i: (0, 0)),
            pl.BlockSpec((PAD, PAD), lambda i: (0, 0)),
        ],
        out_specs=pl.BlockSpec((TBLK, PAD), lambda i: (i, 0)),
        out_shape=jax.ShapeDtypeStruct((V, PAD), jnp.float32),
    )(tT, w1p, w2p)


@functools.partial(
    pl.kernel,
    out_type=jax.ShapeDtypeStruct((B, PAD), jnp.float32),
    mesh=_mesh,
    scratch_types=[
        pltpu.VMEM((2, SB_ROWS, GW), jnp.int32),
        pltpu.VMEM((2, EX_BLK * L, PAD), jnp.float32),
        pltpu.VMEM((2, EX_BLK, PAD), jnp.float32),
        pltpu.VMEM((PAD,), jnp.float32),
        pltpu.SemaphoreType.DMA,
        pltpu.SemaphoreType.DMA,
        pltpu.SemaphoreType.DMA,
        pltpu.SemaphoreType.DMA,
        pltpu.SemaphoreType.DMA,
        pltpu.SemaphoreType.DMA,
    ],
    compiler_params=pltpu.CompilerParams(use_tc_tiling_on_sc=False),
)
def _pool(x_hbm, p_hbm, b_hbm, z_hbm, idx_v, rows_v, zstage, bias_v,
          isem0, isem1, gsem0, gsem1, zsem0, zsem1):
    wid = lax.axis_index("c") * NS + lax.axis_index("s")
    isems = (isem0, isem1)
    gsems = (gsem0, gsem1)
    zsems = (zsem0, zsem1)

    def fire(ib, q, p):
        for j in range(NG):
            pltpu.async_copy(p_hbm.at[idx_v.at[ib, q * NG + j]],
                             rows_v.at[p, pl.ds(j * GW, GW)], gsems[p])

    def drain(ib, q, p):
        for j in range(NG):
            pltpu.make_async_copy(p_hbm.at[idx_v.at[ib, q * NG + j]],
                                  rows_v.at[p, pl.ds(j * GW, GW)],
                                  gsems[p]).wait()

    pltpu.sync_copy(b_hbm, bias_v)
    pltpu.sync_copy(x_hbm.at[wid * NSB], idx_v.at[0])
    fire(0, 0, 0)

    def outer(hh, carry):
        for ib in (0, 1):
            sb = hh * 2 + ib
            nib = 1 - ib

            @pl.when(sb + 1 < NSB)
            def _():
                pltpu.async_copy(x_hbm.at[wid * NSB + sb + 1],
                                 idx_v.at[nib], isems[nib])

            for q in range(SB_BLKS):
                p = q % 2
                np_ = 1 - p
                if q + 1 < SB_BLKS:
                    fire(ib, q + 1, np_)
                else:
                    @pl.when(sb + 1 < NSB)
                    def _():
                        pltpu.make_async_copy(
                            x_hbm.at[wid * NSB + sb + 1], idx_v.at[nib],
                            isems[nib]).wait()
                        fire(nib, 0, np_)
                drain(ib, q, p)

                zero = jnp.zeros((PAD,), jnp.float32)
                bias = bias_v[...]

                def example(r, c):
                    base = r * L

                    def acc_body(i, accs):
                        a0, a1, a2, a3 = accs
                        row = base + i * 4
                        a0 = a0 + rows_v[p, row, pl.ds(0, PAD)]
                        a1 = a1 + rows_v[p, row + 1, pl.ds(0, PAD)]
                        a2 = a2 + rows_v[p, row + 2, pl.ds(0, PAD)]
                        a3 = a3 + rows_v[p, row + 3, pl.ds(0, PAD)]
                        return (a0, a1, a2, a3)

                    a0, a1, a2, a3 = lax.fori_loop(
                        0, L // 4, acc_body, (zero, zero, zero, zero))
                    zstage[p, r, pl.ds(0, PAD)] = ((a0 + a1) + (a2 + a3)
                                                   ) + bias
                    return c

                blk_id = sb * SB_BLKS + q
                row0 = wid * ROWS_W + sb * SB_EX + q * EX_BLK

                @pl.when(blk_id >= 2)
                def _():
                    pltpu.make_async_copy(
                        zstage.at[p],
                        z_hbm.at[pl.ds(row0 - 2 * EX_BLK, EX_BLK)],
                        zsems[p]).wait()

                lax.fori_loop(0, EX_BLK, example, 0)
                pltpu.async_copy(zstage.at[p],
                                 z_hbm.at[pl.ds(row0, EX_BLK)], zsems[p])
        return carry

    lax.fori_loop(0, NSB // 2, outer, 0)
    last = wid * ROWS_W + ROWS_W - EX_BLK
    pltpu.make_async_copy(zstage.at[1],
                          z_hbm.at[pl.ds(last, EX_BLK)], zsems[1]).wait()
    pltpu.make_async_copy(zstage.at[0],
                          z_hbm.at[pl.ds(last - EX_BLK, EX_BLK)],
                          zsems[0]).wait()


def kernel(x, table, W1, b1, W2, b2):
    w1p = jnp.pad(W1, ((0, 0), (0, PAD - W1.shape[1])))
    w2p = jnp.pad(W2, ((0, PAD - W2.shape[0]), (0, PAD - W2.shape[1])))
    bias16 = jnp.pad(jnp.dot(b1, W2) + b2, (0, PAD - W2.shape[1]))
    proj = _project(table.T, w1p, w2p)
    x3 = x.astype(jnp.int32).reshape(B // SB_EX, SB_ROWS, GW)
    z = _pool(x3, proj, bias16)
    return z[:, : W2.shape[1]]
